# baseline (speedup 1.0000x reference)
import dataclasses
import functools
import jax
import jax.numpy as jnp
from jax import lax
from jax.experimental import pallas as pl
from jax.experimental.pallas import tpu as pltpu
from jax.experimental.pallas import tpu_sc as plsc

H = 16
D = 64
L = 8192
HID = 1024
NH = 2
BK = 64
NB = L // BK
NCH = NH * NB
RB = 1024
NRB = L // RB
CPB = RB // BK
SCALE = D ** -0.5


def _proj_body(x_ref, w_ref, b_ref, rot_ref, qv_ref, bkt_ref):
    xb = x_ref[...]
    w = w_ref[0]
    y = jnp.dot(xb, w, preferred_element_type=jnp.float32) + b_ref[0]
    qv_ref[0] = y
    r = jnp.dot(y[:, :D], rot_ref[...], preferred_element_type=jnp.float32)
    iof = lax.broadcasted_iota(jnp.int32, (RB, NB), 1).astype(jnp.float32)
    for a in range(NH):
        ra = r[:, a * 64:(a + 1) * 64]
        full = jnp.concatenate([ra, -ra], axis=1)
        mx = jnp.max(full, axis=1, keepdims=True)
        idxf = jnp.min(jnp.where(full == mx, iof, float(NB)),
                       axis=1, keepdims=True)
        bkt_ref[0, a] = idxf.astype(jnp.int32)


def _run_proj(x2, Wcat, bcat, rot):
    return pl.pallas_call(
        _proj_body,
        compiler_params=pltpu.CompilerParams(
            dimension_semantics=("parallel", "parallel")),
        grid=(NRB, H),
        in_specs=[
            pl.BlockSpec((RB, HID), lambda i, h: (i, 0)),
            pl.BlockSpec((1, HID, 2 * D), lambda i, h: (h, 0, 0)),
            pl.BlockSpec((1, 1, 2 * D), lambda i, h: (h, 0, 0)),
            pl.BlockSpec((D, NH * 64), lambda i, h: (0, 0)),
        ],
        out_specs=[
            pl.BlockSpec((1, RB, 2 * D), lambda i, h: (h, i, 0)),
            pl.BlockSpec((1, NH, RB, 1), lambda i, h: (h, 0, i, 0)),
        ],
        out_shape=[
            jax.ShapeDtypeStruct((H, L, 2 * D), jnp.float32),
            jax.ShapeDtypeStruct((H, NH, L, 1), jnp.int32),
        ],
    )(x2, Wcat, bcat, rot)


_SC_MESH = plsc.VectorSubcoreMesh(core_axis_name="c", subcore_axis_name="s")
_SC_PARAMS = dataclasses.replace(pltpu.CompilerParams(),
                                 needs_layout_passes=False)
_CH = 128
_NCHK = L // _CH
_I16 = lambda: lax.iota(jnp.int32, 16)


def _sort_gather(bkt2d, qv):
    @functools.partial(
        pl.kernel, mesh=_SC_MESH, compiler_params=_SC_PARAMS,
        out_type=[
            jax.ShapeDtypeStruct((NH * H, L), jnp.int32),
            jax.ShapeDtypeStruct((H, NH * L, 2 * D), jnp.float32),
        ],
        scratch_types=[
            pltpu.VMEM((L,), jnp.int32),
            pltpu.VMEM((L,), jnp.int32),
            pltpu.VMEM((NB,), jnp.int32),
            pltpu.VMEM((NB,), jnp.int32),
            pltpu.VMEM((2, _CH, 2 * D), jnp.float32),
            pltpu.SemaphoreType.DMA,
            pltpu.SemaphoreType.DMA,
            pltpu.SemaphoreType.DMA,
            pltpu.SemaphoreType.DMA,
        ],
    )
    def k(bkt_hbm, qv_hbm, st_hbm, sqv_hbm,
          bvm, stv, cnt, run, qbuf, s0, s1, w0, w1):
        h = lax.axis_index("s")
        a = lax.axis_index("c")
        wid = h * NH + a
        pltpu.sync_copy(bkt_hbm.at[wid], bvm)

        z16 = jnp.zeros((16,), jnp.int32)

        @pl.loop(0, NB, step=16)
        def _(o):
            cnt[pl.ds(o, 16)] = z16

        @pl.loop(0, L, step=16)
        def _(o):
            vec = bvm[pl.ds(o, 16)]
            c, last = plsc.scan_count(vec)
            prior = plsc.load_gather(cnt, [vec])
            plsc.store_scatter(cnt, [vec], prior + c, mask=last)

        def _psum(i, carry):
            seg = cnt[pl.ds(i * 16, 16)]
            cs = plsc.cumsum(seg)
            run[pl.ds(i * 16, 16)] = cs - seg + carry
            return carry + jnp.sum(seg)

        lax.fori_loop(0, NB // 16, _psum, jnp.int32(0))

        @pl.loop(0, L, step=16)
        def _(o):
            vec = bvm[pl.ds(o, 16)]
            c, last = plsc.scan_count(vec)
            base = plsc.load_gather(run, [vec])
            plsc.store_scatter(run, [vec], base + c, mask=last)
            plsc.store_scatter(stv, [base + c - 1], _I16() + o)

        pltpu.sync_copy(stv, st_hbm.at[wid])

        @pl.loop(0, _NCHK, step=2)
        def _(ci):
            g = []
            for b in range(2):
                idx = stv.at[pl.ds((ci + b) * _CH, _CH)]
                sem = s0 if b == 0 else s1
                g.append(pltpu.async_copy(qv_hbm.at[h].at[idx], qbuf.at[b], sem))
            wr = []
            for b in range(2):
                g[b].wait()
                dsl = pl.ds(a * L + (ci + b) * _CH, _CH)
                sem = w0 if b == 0 else w1
                wr.append(pltpu.async_copy(qbuf.at[b], sqv_hbm.at[h].at[dsl], sem))
            for c_ in wr:
                c_.wait()

    return k(bkt2d, qv)


def _scatter_back(so, st32):
    @functools.partial(
        pl.kernel, mesh=_SC_MESH, compiler_params=_SC_PARAMS,
        out_type=jax.ShapeDtypeStruct((L * NH * H, 2 * D), jnp.float32),
        scratch_types=[
            pltpu.VMEM((L,), jnp.int32),
            pltpu.VMEM((_NCHK, _CH), jnp.int32),
            pltpu.VMEM((2, _CH, 2 * D), jnp.float32),
            pltpu.SemaphoreType.DMA,
            pltpu.SemaphoreType.DMA,
            pltpu.SemaphoreType.DMA,
            pltpu.SemaphoreType.DMA,
        ],
    )
    def k(so_hbm, st_hbm, ou_hbm,
          stv, idxs, obuf, s0, s1, w0, w1):
        h = lax.axis_index("s")
        a = lax.axis_index("c")
        wid = h * NH + a
        w2 = a * H + h
        pltpu.sync_copy(st_hbm.at[wid], stv)

        @pl.loop(0, _NCHK)
        def _(r):
            for kk in range(_CH // 16):
                vec = stv[pl.ds(r * _CH + kk * 16, 16)]
                idxs[r, pl.ds(kk * 16, 16)] = vec * (NH * H) + w2

        @pl.loop(0, _NCHK, step=2)
        def _(ci):
            g = []
            for b in range(2):
                src = pl.ds(a * L + (ci + b) * _CH, _CH)
                sem = s0 if b == 0 else s1
                g.append(pltpu.async_copy(so_hbm.at[h].at[src], obuf.at[b], sem))
            wr = []
            for b in range(2):
                g[b].wait()
                sem = w0 if b == 0 else w1
                wr.append(pltpu.async_copy(obuf.at[b],
                                           ou_hbm.at[idxs.at[ci + b]], sem))
            for c_ in wr:
                c_.wait()

    return k(so, st32)


BRB = 2048
BCPB = BRB // BK


def _att_body(sqv_ref, sqvp_ref, stc_ref, wnd_ref, so_ref):
    blk = sqv_ref[0]
    pblk = sqvp_ref[0]
    q = blk[:, :D]
    kall = jnp.concatenate([pblk[:, :D], q], axis=0)
    nrm = jnp.sqrt(jnp.sum(kall * kall, axis=1, keepdims=True))
    kn = kall / (nrm + 1e-9)
    vall = jnp.concatenate([pblk[:, D:], blk[:, D:]], axis=0)
    tq = stc_ref[0]
    wnd = wnd_ref[0]
    tk = jnp.broadcast_to(wnd[:, None, :], (BCPB, BK, 2 * BK)).reshape(
        BRB, 2 * BK)
    q16 = q.astype(jnp.bfloat16)
    kn16 = kn.astype(jnp.bfloat16)
    dots = jnp.concatenate(
        [lax.dot_general(q16[j * BK:(j + 1) * BK], kn16[j * BK:(j + 2) * BK],
                         (((1,), (1,)), ((), ())),
                         preferred_element_type=jnp.float32)
         for j in range(BCPB)], axis=0)
    dots = dots * SCALE + jnp.where(tq == tk, -1e5, 0.0)
    m = jnp.max(dots, axis=1, keepdims=True)
    p = jnp.exp(dots - m)
    s = jnp.sum(p, axis=1, keepdims=True)
    pn = p * (1.0 / s)
    lg = m + jnp.log(s)
    pn16 = pn.astype(jnp.bfloat16)
    vall16 = vall.astype(jnp.bfloat16)
    o = jnp.concatenate(
        [jnp.dot(pn16[j * BK:(j + 1) * BK], vall16[j * BK:(j + 2) * BK],
                 preferred_element_type=jnp.float32)
         for j in range(BCPB)], axis=0)
    zpad = jnp.zeros((BRB, D - 1), jnp.float32)
    so_ref[0] = jnp.concatenate([o, lg, zpad], axis=1)


def _run_att(sqv, st_flat):
    nblk = NH * L // BRB
    sqvc = sqv.reshape(H * NCH, BK, 2 * D)
    stc = st_flat.reshape(H, NH * L, 1)
    stc4 = st_flat.reshape(H, NCH, BK)
    wnd = jnp.concatenate([jnp.roll(stc4, 1, axis=1), stc4], axis=-1)

    def pidx(h, i):
        return h * NCH + (i * BCPB - 1) % NCH

    return pl.pallas_call(
        _att_body,
        compiler_params=pltpu.CompilerParams(
            dimension_semantics=("parallel", "parallel")),
        grid=(H, nblk),
        in_specs=[
            pl.BlockSpec((1, BRB, 2 * D), lambda h, i: (h, i, 0)),
            pl.BlockSpec((1, BK, 2 * D), lambda h, i: (pidx(h, i), 0, 0)),
            pl.BlockSpec((1, BRB, 1), lambda h, i: (h, i, 0)),
            pl.BlockSpec((1, BCPB, 2 * BK), lambda h, i: (h, i, 0)),
        ],
        out_specs=[
            pl.BlockSpec((1, BRB, 2 * D), lambda h, i: (h, i, 0)),
        ],
        out_shape=[
            jax.ShapeDtypeStruct((H, NH * L, 2 * D), jnp.float32),
        ],
    )(sqv, sqvc, stc, wnd)


CRB = 512
CCB = 512


def _comb_body(ou_ref, wo_ref, bo_ref, out_ref):
    ou = ou_ref[...]
    lg = ou[:, :, D:D + 1]
    e0 = lg[:, :H, :]
    e1 = lg[:, H:, :]
    m = jnp.maximum(e0, e1)
    p0 = jnp.exp(e0 - m)
    p1 = jnp.exp(e1 - m)
    s = p0 + p1
    wgt = jnp.concatenate([p0 / s, p1 / s], axis=1)
    wb = jnp.broadcast_to(wgt, (CRB, NH * H, 2 * D))
    att = (ou * wb).reshape(CRB, NH * H * 2 * D)
    out_ref[...] = (jnp.dot(att, wo_ref[...], preferred_element_type=jnp.float32)
                    + bo_ref[...])


def _run_comb(ou3, Wo4, bo2):
    return pl.pallas_call(
        _comb_body,
        compiler_params=pltpu.CompilerParams(
            dimension_semantics=("parallel", "parallel")),
        grid=(L // CRB, HID // CCB),
        in_specs=[
            pl.BlockSpec((CRB, NH * H, 2 * D), lambda i, j: (i, 0, 0)),
            pl.BlockSpec((NH * H * 2 * D, CCB), lambda i, j: (0, j)),
            pl.BlockSpec((1, CCB), lambda i, j: (0, j)),
        ],
        out_specs=pl.BlockSpec((CRB, CCB), lambda i, j: (i, j)),
        out_shape=jax.ShapeDtypeStruct((L, HID), jnp.float32),
    )(ou3, Wo4, bo2)


def kernel(x, padding_mask, Wqk, bqk, Wv, bv, Wo, bo):
    del padding_mask
    x2 = x.reshape(L, HID)
    Wcat = jnp.concatenate([Wqk, Wv], axis=-1).transpose(1, 0, 2)
    bcat = jnp.concatenate([bqk, bv], axis=-1).reshape(H, 1, 2 * D)
    rot = jax.random.normal(jax.random.key(42), (D, NH, NB // 2),
                            dtype=jnp.float32).reshape(D, NH * 64)

    qv, bkt = _run_proj(x2, Wcat, bcat, rot)

    st32, sqv = _sort_gather(bkt.reshape(NH * H, L), qv)
    st_flat = st32.reshape(H, NH * L)

    (so,) = _run_att(sqv, st_flat)

    ouf = _scatter_back(so, st32)
    ou3 = ouf.reshape(L, NH * H, 2 * D)

    Wo_flat = Wo.reshape(HID, HID)
    Wo4 = jnp.zeros((NH, H, 2 * D, HID), jnp.float32)
    Wo4 = Wo4.at[:, :, :D, :].set(Wo.reshape(H, D, HID)[None])
    Wo4 = Wo4.reshape(NH * H * 2 * D, HID)
    del Wo_flat
    out = _run_comb(ou3, Wo4, bo.reshape(1, HID))
    return out.reshape(1, L, HID)


# speedup vs baseline: 1.0333x; 1.0333x over previous
import dataclasses
import functools
import jax
import jax.numpy as jnp
from jax import lax
from jax.experimental import pallas as pl
from jax.experimental.pallas import tpu as pltpu
from jax.experimental.pallas import tpu_sc as plsc

H = 16
D = 64
L = 8192
HID = 1024
NH = 2
BK = 64
NB = L // BK
NCH = NH * NB
RB = 1024
NRB = L // RB
CPB = RB // BK
SCALE = D ** -0.5


def _proj_body(x_ref, w_ref, b_ref, rot_ref, qv_ref, bkt_ref):
    xb = x_ref[...]
    w = w_ref[0]
    y = jnp.dot(xb, w, preferred_element_type=jnp.float32) + b_ref[0]
    qv_ref[0] = y
    r = jnp.dot(y[:, :D], rot_ref[...], preferred_element_type=jnp.float32)
    iof = lax.broadcasted_iota(jnp.int32, (RB, NB), 1).astype(jnp.float32)
    for a in range(NH):
        ra = r[:, a * 64:(a + 1) * 64]
        full = jnp.concatenate([ra, -ra], axis=1)
        mx = jnp.max(full, axis=1, keepdims=True)
        idxf = jnp.min(jnp.where(full == mx, iof, float(NB)),
                       axis=1, keepdims=True)
        bkt_ref[0, a] = idxf.astype(jnp.int32)


def _run_proj(x2, Wcat, bcat, rot):
    return pl.pallas_call(
        _proj_body,
        compiler_params=pltpu.CompilerParams(
            dimension_semantics=("parallel", "parallel")),
        grid=(NRB, H),
        in_specs=[
            pl.BlockSpec((RB, HID), lambda i, h: (i, 0)),
            pl.BlockSpec((1, HID, 2 * D), lambda i, h: (h, 0, 0)),
            pl.BlockSpec((1, 1, 2 * D), lambda i, h: (h, 0, 0)),
            pl.BlockSpec((D, NH * 64), lambda i, h: (0, 0)),
        ],
        out_specs=[
            pl.BlockSpec((1, RB, 2 * D), lambda i, h: (h, i, 0)),
            pl.BlockSpec((1, NH, RB, 1), lambda i, h: (h, 0, i, 0)),
        ],
        out_shape=[
            jax.ShapeDtypeStruct((H, L, 2 * D), jnp.float32),
            jax.ShapeDtypeStruct((H, NH, L, 1), jnp.int32),
        ],
    )(x2, Wcat, bcat, rot)


_SC_MESH = plsc.VectorSubcoreMesh(core_axis_name="c", subcore_axis_name="s")
_SC_PARAMS = dataclasses.replace(pltpu.CompilerParams(),
                                 needs_layout_passes=False)
_CH = 128
_NCHK = L // _CH
_I16 = lambda: lax.iota(jnp.int32, 16)


def _sort_gather(bkt2d, qv):
    @functools.partial(
        pl.kernel, mesh=_SC_MESH, compiler_params=_SC_PARAMS,
        out_type=[
            jax.ShapeDtypeStruct((NH * H, L), jnp.int32),
            jax.ShapeDtypeStruct((H, NH * L, 2 * D), jnp.float32),
        ],
        scratch_types=[
            pltpu.VMEM((L,), jnp.int32),
            pltpu.VMEM((L,), jnp.int32),
            pltpu.VMEM((NB,), jnp.int32),
            pltpu.VMEM((NB,), jnp.int32),
            pltpu.VMEM((2, _CH, 2 * D), jnp.float32),
            pltpu.SemaphoreType.DMA,
            pltpu.SemaphoreType.DMA,
            pltpu.SemaphoreType.DMA,
            pltpu.SemaphoreType.DMA,
        ],
    )
    def k(bkt_hbm, qv_hbm, st_hbm, sqv_hbm,
          bvm, stv, cnt, run, qbuf, s0, s1, w0, w1):
        h = lax.axis_index("s")
        a = lax.axis_index("c")
        wid = h * NH + a
        pltpu.sync_copy(bkt_hbm.at[wid], bvm)

        z16 = jnp.zeros((16,), jnp.int32)

        @pl.loop(0, NB, step=16)
        def _(o):
            cnt[pl.ds(o, 16)] = z16

        @pl.loop(0, L, step=16)
        def _(o):
            vec = bvm[pl.ds(o, 16)]
            c, last = plsc.scan_count(vec)
            prior = plsc.load_gather(cnt, [vec])
            plsc.store_scatter(cnt, [vec], prior + c, mask=last)

        def _psum(i, carry):
            seg = cnt[pl.ds(i * 16, 16)]
            cs = plsc.cumsum(seg)
            run[pl.ds(i * 16, 16)] = cs - seg + carry
            return carry + jnp.sum(seg)

        lax.fori_loop(0, NB // 16, _psum, jnp.int32(0))

        @pl.loop(0, L, step=16)
        def _(o):
            vec = bvm[pl.ds(o, 16)]
            c, last = plsc.scan_count(vec)
            base = plsc.load_gather(run, [vec])
            plsc.store_scatter(run, [vec], base + c, mask=last)
            plsc.store_scatter(stv, [base + c - 1], _I16() + o)

        pltpu.sync_copy(stv, st_hbm.at[wid])

        @pl.loop(0, _NCHK, step=2)
        def _(ci):
            g = []
            for b in range(2):
                idx = stv.at[pl.ds((ci + b) * _CH, _CH)]
                sem = s0 if b == 0 else s1
                g.append(pltpu.async_copy(qv_hbm.at[h].at[idx], qbuf.at[b], sem))
            wr = []
            for b in range(2):
                g[b].wait()
                dsl = pl.ds(a * L + (ci + b) * _CH, _CH)
                sem = w0 if b == 0 else w1
                wr.append(pltpu.async_copy(qbuf.at[b], sqv_hbm.at[h].at[dsl], sem))
            for c_ in wr:
                c_.wait()

    return k(bkt2d, qv)


def _scatter_back(so, st32):
    @functools.partial(
        pl.kernel, mesh=_SC_MESH, compiler_params=_SC_PARAMS,
        out_type=jax.ShapeDtypeStruct((L * NH * H, 2 * D), jnp.float32),
        scratch_types=[
            pltpu.VMEM((L,), jnp.int32),
            pltpu.VMEM((_NCHK, _CH), jnp.int32),
            pltpu.VMEM((2, _CH, 2 * D), jnp.float32),
            pltpu.SemaphoreType.DMA,
            pltpu.SemaphoreType.DMA,
            pltpu.SemaphoreType.DMA,
            pltpu.SemaphoreType.DMA,
        ],
    )
    def k(so_hbm, st_hbm, ou_hbm,
          stv, idxs, obuf, s0, s1, w0, w1):
        h = lax.axis_index("s")
        a = lax.axis_index("c")
        wid = h * NH + a
        w2 = a * H + h
        pltpu.sync_copy(st_hbm.at[wid], stv)

        @pl.loop(0, _NCHK)
        def _(r):
            for kk in range(_CH // 16):
                vec = stv[pl.ds(r * _CH + kk * 16, 16)]
                idxs[r, pl.ds(kk * 16, 16)] = vec * (NH * H) + w2

        @pl.loop(0, _NCHK, step=2)
        def _(ci):
            g = []
            for b in range(2):
                src = pl.ds(a * L + (ci + b) * _CH, _CH)
                sem = s0 if b == 0 else s1
                g.append(pltpu.async_copy(so_hbm.at[h].at[src], obuf.at[b], sem))
            wr = []
            for b in range(2):
                g[b].wait()
                sem = w0 if b == 0 else w1
                wr.append(pltpu.async_copy(obuf.at[b],
                                           ou_hbm.at[idxs.at[ci + b]], sem))
            for c_ in wr:
                c_.wait()

    return k(so, st32)


BRB = 2048
BCPB = BRB // BK


def _att_body(sqv_ref, sqvp_ref, stc_ref, wnd_ref, so_ref):
    blk = sqv_ref[0]
    pblk = sqvp_ref[0]
    q = blk[:, :D]
    kall = jnp.concatenate([pblk[:, :D], q], axis=0)
    nrm = jnp.sqrt(jnp.sum(kall * kall, axis=1, keepdims=True))
    kn = kall / (nrm + 1e-9)
    vall = jnp.concatenate([pblk[:, D:], blk[:, D:]], axis=0)
    tq = stc_ref[0]
    wnd = wnd_ref[0]
    tk = jnp.broadcast_to(wnd[:, None, :], (BCPB, BK, 2 * BK)).reshape(
        BRB, 2 * BK)
    dots = jnp.concatenate(
        [lax.dot_general(q[j * BK:(j + 1) * BK], kn[j * BK:(j + 2) * BK],
                         (((1,), (1,)), ((), ())),
                         preferred_element_type=jnp.float32)
         for j in range(BCPB)], axis=0)
    dots = dots * SCALE + jnp.where(tq == tk, -1e5, 0.0)
    m = jnp.max(dots, axis=1, keepdims=True)
    p = jnp.exp(dots - m)
    s = jnp.sum(p, axis=1, keepdims=True)
    pn = p * (1.0 / s)
    lg = m + jnp.log(s)
    o = jnp.concatenate(
        [jnp.dot(pn[j * BK:(j + 1) * BK], vall[j * BK:(j + 2) * BK],
                 preferred_element_type=jnp.float32)
         for j in range(BCPB)], axis=0)
    zpad = jnp.zeros((BRB, D - 1), jnp.float32)
    so_ref[0] = jnp.concatenate([o, lg, zpad], axis=1)


def _run_att(sqv, st_flat):
    nblk = NH * L // BRB
    sqvc = sqv.reshape(H * NCH, BK, 2 * D)
    stc = st_flat.reshape(H, NH * L, 1)
    stc4 = st_flat.reshape(H, NCH, BK)
    wnd = jnp.concatenate([jnp.roll(stc4, 1, axis=1), stc4], axis=-1)

    def pidx(h, i):
        return h * NCH + (i * BCPB - 1) % NCH

    return pl.pallas_call(
        _att_body,
        compiler_params=pltpu.CompilerParams(
            dimension_semantics=("parallel", "parallel")),
        grid=(H, nblk),
        in_specs=[
            pl.BlockSpec((1, BRB, 2 * D), lambda h, i: (h, i, 0)),
            pl.BlockSpec((1, BK, 2 * D), lambda h, i: (pidx(h, i), 0, 0)),
            pl.BlockSpec((1, BRB, 1), lambda h, i: (h, i, 0)),
            pl.BlockSpec((1, BCPB, 2 * BK), lambda h, i: (h, i, 0)),
        ],
        out_specs=[
            pl.BlockSpec((1, BRB, 2 * D), lambda h, i: (h, i, 0)),
        ],
        out_shape=[
            jax.ShapeDtypeStruct((H, NH * L, 2 * D), jnp.float32),
        ],
    )(sqv, sqvc, stc, wnd)


CRB = 512
CCB = 512


def _comb_body(ou_ref, wo_ref, bo_ref, out_ref):
    ou = ou_ref[...]
    lg = ou[:, :, D:D + 1]
    e0 = lg[:, :H, :]
    e1 = lg[:, H:, :]
    m = jnp.maximum(e0, e1)
    p0 = jnp.exp(e0 - m)
    p1 = jnp.exp(e1 - m)
    s = p0 + p1
    wgt = jnp.concatenate([p0 / s, p1 / s], axis=1)
    wb = jnp.broadcast_to(wgt, (CRB, NH * H, 2 * D))
    att = (ou * wb).reshape(CRB, NH * H * 2 * D)
    out_ref[...] = (jnp.dot(att, wo_ref[...], preferred_element_type=jnp.float32)
                    + bo_ref[...])


def _run_comb(ou3, Wo4, bo2):
    return pl.pallas_call(
        _comb_body,
        compiler_params=pltpu.CompilerParams(
            dimension_semantics=("parallel", "parallel")),
        grid=(L // CRB, HID // CCB),
        in_specs=[
            pl.BlockSpec((CRB, NH * H, 2 * D), lambda i, j: (i, 0, 0)),
            pl.BlockSpec((NH * H * 2 * D, CCB), lambda i, j: (0, j)),
            pl.BlockSpec((1, CCB), lambda i, j: (0, j)),
        ],
        out_specs=pl.BlockSpec((CRB, CCB), lambda i, j: (i, j)),
        out_shape=jax.ShapeDtypeStruct((L, HID), jnp.float32),
    )(ou3, Wo4, bo2)


def kernel(x, padding_mask, Wqk, bqk, Wv, bv, Wo, bo):
    del padding_mask
    x2 = x.reshape(L, HID)
    Wcat = jnp.concatenate([Wqk, Wv], axis=-1).transpose(1, 0, 2)
    bcat = jnp.concatenate([bqk, bv], axis=-1).reshape(H, 1, 2 * D)
    rot = jax.random.normal(jax.random.key(42), (D, NH, NB // 2),
                            dtype=jnp.float32).reshape(D, NH * 64)

    qv, bkt = _run_proj(x2, Wcat, bcat, rot)

    st32, sqv = _sort_gather(bkt.reshape(NH * H, L), qv)
    st_flat = st32.reshape(H, NH * L)

    (so,) = _run_att(sqv, st_flat)

    ouf = _scatter_back(so, st32)
    ou3 = ouf.reshape(L, NH * H, 2 * D)

    Wo_flat = Wo.reshape(HID, HID)
    Wo4 = jnp.zeros((NH, H, 2 * D, HID), jnp.float32)
    Wo4 = Wo4.at[:, :, :D, :].set(Wo.reshape(H, D, HID)[None])
    Wo4 = Wo4.reshape(NH * H * 2 * D, HID)
    del Wo_flat
    out = _run_comb(ou3, Wo4, bo.reshape(1, HID))
    return out.reshape(1, L, HID)
